# baseline (speedup 1.0000x reference)
import functools

import jax
import jax.numpy as jnp
from jax import lax
from jax.experimental import pallas as pl
from jax.experimental.pallas import tpu as pltpu
from jax.experimental.pallas import tpu_sc as plsc

N = 100000
D = 128
G = 512
C = 10
NC, NS = 2, 16
NW = NC * NS
CHUNK = 128
NFULL = N // CHUNK
TAIL = N - NFULL * CHUNK
TAIL_BASE = NFULL * CHUNK
CPW = -(-NFULL // NW)
NTRI = (CPW - 1) // 3
G_PER_TILE = G // NS

_mesh = plsc.VectorSubcoreMesh(core_axis_name="c", subcore_axis_name="s",
                               num_cores=NC, num_subcores=NS)


@functools.partial(
    pl.kernel,
    out_type=jax.ShapeDtypeStruct((NC * G, D), jnp.float32),
    mesh=_mesh,
    scratch_types=[
        pltpu.VMEM((CHUNK,), jnp.int32),
        pltpu.VMEM((CHUNK,), jnp.int32),
        pltpu.VMEM((CHUNK,), jnp.int32),
        pltpu.VMEM((CHUNK, D), jnp.float32),
        pltpu.VMEM((CHUNK, D), jnp.float32),
        pltpu.VMEM((CHUNK, D), jnp.float32),
        pltpu.VMEM((TAIL,), jnp.int32),
        pltpu.VMEM((TAIL, D), jnp.float32),
        pltpu.VMEM_SHARED((G, D), jnp.float32),
        pltpu.SemaphoreType.DMA,
        pltpu.SemaphoreType.DMA,
        pltpu.SemaphoreType.DMA,
        pltpu.SemaphoreType.DMA,
        pltpu.SemaphoreType.DMA,
        pltpu.SemaphoreType.DMA,
    ],
)
def _sc_pool(x_hbm, b_hbm, out_hbm, idx0, idx1, idx2, rows0, rows1, rows2,
             idxt, rowst, acc_sh, sg0, sg1, sg2, ss0, ss1, ss2):
    cid = lax.axis_index("c")
    sid = lax.axis_index("s")
    wid = cid * NS + sid

    idxs = (idx0, idx1, idx2)
    bufs = (rows0, rows1, rows2)
    sgs = (sg0, sg1, sg2)
    sss = (ss0, ss1, ss2)

    def issue(ci, b):
        base = ci * CHUNK
        pltpu.async_copy(b_hbm.at[pl.ds(base, CHUNK)], idxs[b], sgs[b])
        pltpu.async_copy(x_hbm.at[pl.ds(base, CHUNK)], bufs[b], sgs[b])

    def wait(ci, b):
        base = ci * CHUNK
        pltpu.make_async_copy(b_hbm.at[pl.ds(base, CHUNK)], idxs[b],
                              sgs[b]).wait()
        pltpu.make_async_copy(x_hbm.at[pl.ds(base, CHUNK)], bufs[b],
                              sgs[b]).wait()

    def scat(b):
        pltpu.async_copy(bufs[b], acc_sh.at[idxs[b]], sss[b], add=True)

    def scat_wait(b):
        pltpu.make_async_copy(bufs[b], acc_sh.at[idxs[b]], sss[b]).wait()

    @pl.loop(0, G_PER_TILE)
    def _(r):
        @pl.loop(0, D // 16)
        def _(c):
            rows0[r, pl.ds(c * 16, 16)] = jnp.zeros((16,), jnp.float32)

    pltpu.sync_copy(rows0.at[pl.ds(0, G_PER_TILE)],
                    acc_sh.at[pl.ds(sid * G_PER_TILE, G_PER_TILE)])
    plsc.subcore_barrier()

    issue(wid, 0)
    issue(wid + NW, 1)

    @pl.loop(0, NTRI)
    def _(t):
        for db in range(3):
            j = 3 * t + db
            ci = wid + j * NW
            wait(ci, db)
            scat(db)
            pb = (db - 1) % 3
            if db == 0:
                @pl.when(t > 0)
                def _():
                    scat_wait(pb)
            else:
                scat_wait(pb)
            nci = ci + 2 * NW

            @pl.when(nci < NFULL)
            def _():
                issue(nci, (db + 2) % 3)

    lci = wid + (CPW - 1) * NW

    @pl.when(lci < NFULL)
    def _():
        wait(lci, 0)
        scat(0)

    scat_wait(2)

    @pl.when(lci < NFULL)
    def _():
        scat_wait(0)

    @pl.when(wid == NW - 1)
    def _():
        pltpu.sync_copy(b_hbm.at[pl.ds(TAIL_BASE, TAIL)], idxt)
        pltpu.sync_copy(x_hbm.at[pl.ds(TAIL_BASE, TAIL)], rowst)
        pltpu.sync_copy(rowst, acc_sh.at[idxt], add=True)

    plsc.subcore_barrier()

    pltpu.sync_copy(acc_sh.at[pl.ds(sid * G_PER_TILE, G_PER_TILE)],
                    out_hbm.at[pl.ds(cid * G + sid * G_PER_TILE, G_PER_TILE)])


def _mlp_body(p_ref, w1_ref, b1_ref, w2_ref, b2_ref, o_ref):
    pooled = p_ref[:G, :] + p_ref[G:, :]
    h = jnp.dot(pooled, w1_ref[...], preferred_element_type=jnp.float32)
    h = jnp.maximum(h + b1_ref[...], 0.0)
    o = jnp.dot(h, w2_ref[...], preferred_element_type=jnp.float32) + b2_ref[...]
    m = jnp.max(o, axis=-1, keepdims=True)
    lse = jnp.log(jnp.sum(jnp.exp(o - m), axis=-1, keepdims=True)) + m
    o_ref[...] = o - lse


_mlp = pl.pallas_call(
    _mlp_body,
    out_shape=jax.ShapeDtypeStruct((G, C), jnp.float32),
)


def kernel(x, batch, W1, b1, W2, b2):
    batch = batch.astype(jnp.int32)
    partials = _sc_pool(x, batch)
    return _mlp(partials, W1, b1[None, :], W2, b2[None, :])
